# baseline (speedup 1.0000x reference)
import functools

import jax
import jax.numpy as jnp
from jax import lax
from jax.experimental import pallas as pl
from jax.experimental.pallas import tpu as pltpu
from jax.experimental.pallas import tpu_sc as plsc

N_NODES = 10000
N_EDGES = 160000
D_IN = 256
D_HID = 1024
D_OUT = 256
BN_EPS = 1e-5

R = 400
NBLK = N_NODES // R


def _phase1_body(eps_ref, x_ref, aggA_ref, aggB_ref, W1_ref, b1_ref,
                 h1_ref, s1_ref, s2_ref):
    i = pl.program_id(0)
    scale = 1.0 + eps_ref[0, 0]
    hinA = scale * x_ref[:, :128] + aggA_ref[...]
    hinB = scale * x_ref[:, 128:] + aggB_ref[...]
    h1 = (jnp.dot(hinA, W1_ref[:128, :], preferred_element_type=jnp.float32)
          + jnp.dot(hinB, W1_ref[128:, :], preferred_element_type=jnp.float32)
          + b1_ref[...])
    h1_ref[...] = h1
    ps1 = jnp.sum(h1, axis=0, keepdims=True)
    ps2 = jnp.sum(h1 * h1, axis=0, keepdims=True)

    @pl.when(i == 0)
    def _():
        s1_ref[...] = ps1
        s2_ref[...] = ps2

    @pl.when(i != 0)
    def _():
        s1_ref[...] += ps1
        s2_ref[...] += ps2


def _phase2_body(h1_ref, s1_ref, s2_ref, g1_ref, beta1_ref, W2_ref, b2_ref,
                 h2_ref, t1_ref, t2_ref):
    i = pl.program_id(0)
    n = jnp.float32(N_NODES)
    mu = s1_ref[...] / n
    var = s2_ref[...] / n - mu * mu
    rstd = lax.rsqrt(var + BN_EPS)
    a = (h1_ref[...] - mu) * (rstd * g1_ref[...]) + beta1_ref[...]
    a = jnp.maximum(a, 0.0)
    h2 = jnp.dot(a, W2_ref[...], preferred_element_type=jnp.float32) + b2_ref[...]
    h2_ref[...] = h2
    ps1 = jnp.sum(h2, axis=0, keepdims=True)
    ps2 = jnp.sum(h2 * h2, axis=0, keepdims=True)

    @pl.when(i == 0)
    def _():
        t1_ref[...] = ps1
        t2_ref[...] = ps2

    @pl.when(i != 0)
    def _():
        t1_ref[...] += ps1
        t2_ref[...] += ps2


def _phase3_body(h2_ref, t1_ref, t2_ref, g2_ref, beta2_ref, out_ref):
    n = jnp.float32(N_NODES)
    mu = t1_ref[...] / n
    var = t2_ref[...] / n - mu * mu
    rstd = lax.rsqrt(var + BN_EPS)
    out_ref[...] = (h2_ref[...] - mu) * (rstd * g2_ref[...]) + beta2_ref[...]


K = 128
E_TILE = 10752
CH = E_TILE // K
N_ACC = 10008
OROWS = 624
NBUF = 2
NIDX = 6


def _agg_sc_body(x2_hbm, idx_hbm, zeros_hbm, out_hbm,
                 idx_vm, rows_vm, acc_sh, gsems, ssems, isems):
    c = lax.axis_index("c")
    s = lax.axis_index("s")
    pltpu.sync_copy(zeros_hbm, acc_sh.at[pl.ds(s * OROWS, OROWS)])

    @pl.when(s == 15)
    def _():
        pltpu.sync_copy(zeros_hbm.at[pl.ds(0, N_ACC - 16 * OROWS)],
                        acc_sh.at[pl.ds(16 * OROWS, N_ACC - 16 * OROWS)])

    plsc.subcore_barrier()

    def idx_load(j, bi):
        pltpu.async_copy(idx_hbm.at[c, s, j], idx_vm.at[bi], isems[bi])

    def wait_idx(bi):
        pltpu.make_async_copy(idx_hbm.at[c, s, 0], idx_vm.at[bi],
                              isems[bi]).wait()

    def gather(b, bi):
        pltpu.async_copy(x2_hbm.at[idx_vm.at[bi, 0]], rows_vm.at[b], gsems[b])

    def wait_gather(b, bi):
        pltpu.make_async_copy(x2_hbm.at[idx_vm.at[bi, 0]], rows_vm.at[b],
                              gsems[b]).wait()

    def scatter(b, bi):
        pltpu.async_copy(rows_vm.at[b], acc_sh.at[idx_vm.at[bi, 1]], ssems[b],
                         add=True)

    def wait_scatter(b):
        pltpu.make_async_copy(rows_vm.at[b], acc_sh.at[idx_vm.at[0, 1]],
                              ssems[b]).wait()

    for m in range(4):
        idx_load(m, m)
    wait_idx(0)
    gather(0, 0)

    def body(g, carry):
        for t in range(NIDX):
            j = g * NIDX + t
            b, bn = t % NBUF, (t + 1) % NBUF
            bi, bn6, bp4 = t, (t + 1) % NIDX, (t + 4) % NIDX

            @pl.when(j + 1 < CH)
            def _():
                wait_idx(bn6)
                gather(bn, bn6)

            wait_gather(b, bi)
            scatter(b, bi)
            wait_scatter(b)

            @pl.when(j + 4 < CH)
            def _():
                idx_load(j + 4, bp4)
        return carry

    lax.fori_loop(0, CH // NIDX, body, 0)
    plsc.subcore_barrier()
    pltpu.sync_copy(acc_sh.at[pl.ds(s * OROWS, OROWS)],
                    out_hbm.at[c, pl.ds(s * OROWS, OROWS)])

    @pl.when(s == 15)
    def _():
        pltpu.sync_copy(acc_sh.at[pl.ds(16 * OROWS, N_NODES - 16 * OROWS)],
                        out_hbm.at[c, pl.ds(16 * OROWS, N_NODES - 16 * OROWS)])


@functools.partial(
    pl.kernel,
    out_type=jax.ShapeDtypeStruct((2, N_NODES, 128), jnp.float32),
    mesh=plsc.VectorSubcoreMesh(core_axis_name="c", subcore_axis_name="s"),
    scratch_types=[
        pltpu.VMEM((NIDX, 2, K), jnp.int32),
        pltpu.VMEM((NBUF, K, 128), jnp.float32),
        pltpu.VMEM_SHARED((N_ACC, 128), jnp.float32),
    ] + [pltpu.SemaphoreType.DMA] * (2 * NBUF + NIDX),
)
def _agg_sc(x2_hbm, idx_hbm, zeros_hbm, out_hbm,
            idx_vm, rows_vm, acc_sh, *sems):
    _agg_sc_body(x2_hbm, idx_hbm, zeros_hbm, out_hbm,
                 idx_vm, rows_vm, acc_sh,
                 sems[:NBUF], sems[NBUF:2 * NBUF], sems[2 * NBUF:])


def _aggregate(x, src, dst):
    npad = E_TILE - N_EDGES // 16
    x2 = x.reshape(2 * N_NODES, 128)
    src_t = jnp.concatenate(
        [src.reshape(16, N_EDGES // 16),
         jnp.zeros((16, npad), jnp.int32)], axis=1).reshape(16, CH, K)
    dst_t = jnp.concatenate(
        [dst.reshape(16, N_EDGES // 16),
         jnp.full((16, npad), N_NODES, jnp.int32)], axis=1).reshape(16, CH, K)
    idx = jnp.stack([jnp.stack([src_t * 2, dst_t], axis=2),
                     jnp.stack([src_t * 2 + 1, dst_t], axis=2)])
    zeros = jnp.zeros((OROWS, 128), jnp.float32)
    agg = _agg_sc(x2, idx, zeros)
    return agg[0], agg[1]


def kernel(x, edge_index, eps, W1, b1, g1, beta1, W2, b2, g2, beta2):
    src = edge_index[0].astype(jnp.int32)
    dst = edge_index[1].astype(jnp.int32)
    aggA, aggB = _aggregate(x, src, dst)

    eps2 = eps.reshape(1, 1)
    b1r = b1.reshape(1, D_HID)
    g1r = g1.reshape(1, D_HID)
    beta1r = beta1.reshape(1, D_HID)
    b2r = b2.reshape(1, D_OUT)
    g2r = g2.reshape(1, D_OUT)
    beta2r = beta2.reshape(1, D_OUT)

    full = lambda shape: pl.BlockSpec(shape, lambda i: (0,) * len(shape))
    rowblk = lambda c: pl.BlockSpec((R, c), lambda i: (i, 0))

    h1, s1, s2 = pl.pallas_call(
        _phase1_body,
        grid=(NBLK,),
        in_specs=[full((1, 1)), rowblk(D_IN), rowblk(128), rowblk(128),
                  full((D_IN, D_HID)), full((1, D_HID))],
        out_specs=[rowblk(D_HID), full((1, D_HID)), full((1, D_HID))],
        out_shape=[jax.ShapeDtypeStruct((N_NODES, D_HID), jnp.float32),
                   jax.ShapeDtypeStruct((1, D_HID), jnp.float32),
                   jax.ShapeDtypeStruct((1, D_HID), jnp.float32)],
    )(eps2, x, aggA, aggB, W1, b1r)

    h2, t1, t2 = pl.pallas_call(
        _phase2_body,
        grid=(NBLK,),
        in_specs=[rowblk(D_HID), full((1, D_HID)), full((1, D_HID)),
                  full((1, D_HID)), full((1, D_HID)),
                  full((D_HID, D_OUT)), full((1, D_OUT))],
        out_specs=[rowblk(D_OUT), full((1, D_OUT)), full((1, D_OUT))],
        out_shape=[jax.ShapeDtypeStruct((N_NODES, D_OUT), jnp.float32),
                   jax.ShapeDtypeStruct((1, D_OUT), jnp.float32),
                   jax.ShapeDtypeStruct((1, D_OUT), jnp.float32)],
    )(h1, s1, s2, g1r, beta1r, W2, b2r)

    out = pl.pallas_call(
        _phase3_body,
        grid=(NBLK,),
        in_specs=[rowblk(D_OUT), full((1, D_OUT)), full((1, D_OUT)),
                  full((1, D_OUT)), full((1, D_OUT))],
        out_specs=rowblk(D_OUT),
        out_shape=jax.ShapeDtypeStruct((N_NODES, D_OUT), jnp.float32),
    )(h2, t1, t2, g2r, beta2r)
    return out


# speedup vs baseline: 1.5082x; 1.5082x over previous
import functools

import jax
import jax.numpy as jnp
from jax import lax
from jax.experimental import pallas as pl
from jax.experimental.pallas import tpu as pltpu
from jax.experimental.pallas import tpu_sc as plsc

N_NODES = 10000
N_EDGES = 160000
D_IN = 256
D_HID = 1024
D_OUT = 256
BN_EPS = 1e-5

R = 400
NBLK = N_NODES // R


def _phase1_body(eps_ref, x_ref, aggA_ref, aggB_ref, W1_ref, b1_ref,
                 h1_ref, s1_ref, s2_ref):
    i = pl.program_id(0)
    scale = 1.0 + eps_ref[0, 0]
    hinA = scale * x_ref[:, :128] + aggA_ref[...]
    hinB = scale * x_ref[:, 128:] + aggB_ref[...]
    h1 = (jnp.dot(hinA, W1_ref[:128, :], preferred_element_type=jnp.float32)
          + jnp.dot(hinB, W1_ref[128:, :], preferred_element_type=jnp.float32)
          + b1_ref[...])
    h1_ref[...] = h1
    ps1 = jnp.sum(h1, axis=0, keepdims=True)
    ps2 = jnp.sum(h1 * h1, axis=0, keepdims=True)

    @pl.when(i == 0)
    def _():
        s1_ref[...] = ps1
        s2_ref[...] = ps2

    @pl.when(i != 0)
    def _():
        s1_ref[...] += ps1
        s2_ref[...] += ps2


def _phase2_body(h1_ref, s1_ref, s2_ref, g1_ref, beta1_ref, W2_ref, b2_ref,
                 h2_ref, t1_ref, t2_ref):
    i = pl.program_id(0)
    n = jnp.float32(N_NODES)
    mu = s1_ref[...] / n
    var = s2_ref[...] / n - mu * mu
    rstd = lax.rsqrt(var + BN_EPS)
    a = (h1_ref[...] - mu) * (rstd * g1_ref[...]) + beta1_ref[...]
    a = jnp.maximum(a, 0.0)
    h2 = jnp.dot(a, W2_ref[...], preferred_element_type=jnp.float32) + b2_ref[...]
    h2_ref[...] = h2
    ps1 = jnp.sum(h2, axis=0, keepdims=True)
    ps2 = jnp.sum(h2 * h2, axis=0, keepdims=True)

    @pl.when(i == 0)
    def _():
        t1_ref[...] = ps1
        t2_ref[...] = ps2

    @pl.when(i != 0)
    def _():
        t1_ref[...] += ps1
        t2_ref[...] += ps2


def _phase3_body(h2_ref, t1_ref, t2_ref, g2_ref, beta2_ref, out_ref):
    n = jnp.float32(N_NODES)
    mu = t1_ref[...] / n
    var = t2_ref[...] / n - mu * mu
    rstd = lax.rsqrt(var + BN_EPS)
    out_ref[...] = (h2_ref[...] - mu) * (rstd * g2_ref[...]) + beta2_ref[...]


K = 128
SCH = 8
NSUP = 10
E_TILE = NSUP * SCH * K
N_ACC = 10008
OROWS = 624


def _agg_sc_body(x2_hbm, idx_hbm, zeros_hbm, out_hbm,
                 idx_vm, rows_vm, acc_sh, gsems, ssem, isems):
    c = lax.axis_index("c")
    s = lax.axis_index("s")
    pltpu.sync_copy(zeros_hbm, acc_sh.at[pl.ds(s * OROWS, OROWS)])

    @pl.when(s == 15)
    def _():
        pltpu.sync_copy(zeros_hbm.at[pl.ds(0, N_ACC - 16 * OROWS)],
                        acc_sh.at[pl.ds(16 * OROWS, N_ACC - 16 * OROWS)])

    plsc.subcore_barrier()

    def idx_load(m, bi):
        pltpu.async_copy(idx_hbm.at[c, s, m], idx_vm.at[bi], isems[bi])

    def wait_idx(bi):
        pltpu.make_async_copy(idx_hbm.at[c, s, 0], idx_vm.at[bi],
                              isems[bi]).wait()

    def gather(bi, t, b):
        pltpu.async_copy(x2_hbm.at[idx_vm.at[bi, t, 0]], rows_vm.at[b],
                         gsems[b])

    def wait_gather(b):
        pltpu.make_async_copy(x2_hbm.at[idx_vm.at[0, 0, 0]], rows_vm.at[b],
                              gsems[b]).wait()

    def scatter(bi, t, b):
        pltpu.async_copy(rows_vm.at[b], acc_sh.at[idx_vm.at[bi, t, 1]], ssem,
                         add=True)
        pltpu.make_async_copy(rows_vm.at[b], acc_sh.at[idx_vm.at[bi, t, 1]],
                              ssem).wait()

    def super_body(g, islot):
        idx_load(g + 1, islot ^ 1)
        for t in range(SCH):
            if t < SCH - 1:
                gather(islot, t + 1, (t + 1) & 1)
            else:
                wait_idx(islot ^ 1)
                gather(islot ^ 1, 0, 0)
            wait_gather(t & 1)
            scatter(islot, t, t & 1)

    idx_load(0, 0)
    wait_idx(0)
    gather(0, 0, 0)

    def body(gg, carry):
        super_body(2 * gg, 0)
        super_body(2 * gg + 1, 1)
        return carry

    lax.fori_loop(0, NSUP // 2, body, 0)
    wait_gather(0)
    plsc.subcore_barrier()
    pltpu.sync_copy(acc_sh.at[pl.ds(s * OROWS, OROWS)],
                    out_hbm.at[c, pl.ds(s * OROWS, OROWS)])

    @pl.when(s == 15)
    def _():
        pltpu.sync_copy(acc_sh.at[pl.ds(16 * OROWS, N_NODES - 16 * OROWS)],
                        out_hbm.at[c, pl.ds(16 * OROWS, N_NODES - 16 * OROWS)])


@functools.partial(
    pl.kernel,
    out_type=jax.ShapeDtypeStruct((2, N_NODES, 128), jnp.float32),
    mesh=plsc.VectorSubcoreMesh(core_axis_name="c", subcore_axis_name="s"),
    scratch_types=[
        pltpu.VMEM((2, SCH, 2, K), jnp.int32),
        pltpu.VMEM((2, K, 128), jnp.float32),
        pltpu.VMEM_SHARED((N_ACC, 128), jnp.float32),
    ] + [pltpu.SemaphoreType.DMA] * 5,
)
def _agg_sc(x2_hbm, idx_hbm, zeros_hbm, out_hbm,
            idx_vm, rows_vm, acc_sh, *sems):
    _agg_sc_body(x2_hbm, idx_hbm, zeros_hbm, out_hbm,
                 idx_vm, rows_vm, acc_sh,
                 sems[:2], sems[2], sems[3:])


def _aggregate(x, src, dst):
    e_full = (NSUP + 1) * SCH * K
    npad = e_full - N_EDGES // 16
    x2 = x.reshape(2 * N_NODES, 128)
    src_t = jnp.concatenate(
        [src.reshape(16, N_EDGES // 16),
         jnp.zeros((16, npad), jnp.int32)], axis=1).reshape(16, NSUP + 1, SCH, K)
    dst_t = jnp.concatenate(
        [dst.reshape(16, N_EDGES // 16),
         jnp.full((16, npad), N_NODES, jnp.int32)],
        axis=1).reshape(16, NSUP + 1, SCH, K)
    idx = jnp.stack([jnp.stack([src_t * 2, dst_t], axis=3),
                     jnp.stack([src_t * 2 + 1, dst_t], axis=3)])
    zeros = jnp.zeros((OROWS, 128), jnp.float32)
    agg = _agg_sc(x2, idx, zeros)
    return agg[0], agg[1]


def kernel(x, edge_index, eps, W1, b1, g1, beta1, W2, b2, g2, beta2):
    src = edge_index[0].astype(jnp.int32)
    dst = edge_index[1].astype(jnp.int32)
    aggA, aggB = _aggregate(x, src, dst)

    eps2 = eps.reshape(1, 1)
    b1r = b1.reshape(1, D_HID)
    g1r = g1.reshape(1, D_HID)
    beta1r = beta1.reshape(1, D_HID)
    b2r = b2.reshape(1, D_OUT)
    g2r = g2.reshape(1, D_OUT)
    beta2r = beta2.reshape(1, D_OUT)

    full = lambda shape: pl.BlockSpec(shape, lambda i: (0,) * len(shape))
    rowblk = lambda c: pl.BlockSpec((R, c), lambda i: (i, 0))

    h1, s1, s2 = pl.pallas_call(
        _phase1_body,
        grid=(NBLK,),
        in_specs=[full((1, 1)), rowblk(D_IN), rowblk(128), rowblk(128),
                  full((D_IN, D_HID)), full((1, D_HID))],
        out_specs=[rowblk(D_HID), full((1, D_HID)), full((1, D_HID))],
        out_shape=[jax.ShapeDtypeStruct((N_NODES, D_HID), jnp.float32),
                   jax.ShapeDtypeStruct((1, D_HID), jnp.float32),
                   jax.ShapeDtypeStruct((1, D_HID), jnp.float32)],
    )(eps2, x, aggA, aggB, W1, b1r)

    h2, t1, t2 = pl.pallas_call(
        _phase2_body,
        grid=(NBLK,),
        in_specs=[rowblk(D_HID), full((1, D_HID)), full((1, D_HID)),
                  full((1, D_HID)), full((1, D_HID)),
                  full((D_HID, D_OUT)), full((1, D_OUT))],
        out_specs=[rowblk(D_OUT), full((1, D_OUT)), full((1, D_OUT))],
        out_shape=[jax.ShapeDtypeStruct((N_NODES, D_OUT), jnp.float32),
                   jax.ShapeDtypeStruct((1, D_OUT), jnp.float32),
                   jax.ShapeDtypeStruct((1, D_OUT), jnp.float32)],
    )(h1, s1, s2, g1r, beta1r, W2, b2r)

    out = pl.pallas_call(
        _phase3_body,
        grid=(NBLK,),
        in_specs=[rowblk(D_OUT), full((1, D_OUT)), full((1, D_OUT)),
                  full((1, D_OUT)), full((1, D_OUT))],
        out_specs=rowblk(D_OUT),
        out_shape=jax.ShapeDtypeStruct((N_NODES, D_OUT), jnp.float32),
    )(h2, t1, t2, g2r, beta2r)
    return out
